# baseline (speedup 1.0000x reference)
import functools

import jax
import jax.numpy as jnp
from jax import lax
from jax.experimental import pallas as pl
from jax.experimental.pallas import tpu as pltpu
from jax.experimental.pallas import tpu_sc as plsc

B, S, D = 4, 2048, 2048
HALF = D // 2
NUM_SPANS = 2048
WIDTH_DIM = 64
OUT_D = 2 * HALF + WIDTH_DIM
OUT_D_PAD = 2 * HALF + 128

NW = 32
P = (B * NUM_SPANS) // NW
C = 16
NCHUNK = P // C
WPB = NW // B


def _sc_body(seq2, starts, ends, sent_s_h, sent_e_h, wemb_h, out,
             fs_buf, fe_buf, bx_buf, bs_buf, wemb_buf,
             sent_s, sent_e, st_ref, en_ref,
             fs_idx, fe_idx, bx_idx, bs_idx, ms_ref, me_ref, wb_idx,
             sem_g, sem_w):
    c_id = lax.axis_index("c")
    s_id = lax.axis_index("s")
    wid = s_id * 2 + c_id
    r0 = pl.multiple_of(wid * P, P)
    base2 = jnp.full((16,), (wid // WPB) * (2 * S), jnp.int32)

    pltpu.sync_copy(starts.at[pl.ds(r0, P)], st_ref)
    pltpu.sync_copy(ends.at[pl.ds(r0, P)], en_ref)
    pltpu.sync_copy(sent_s_h, sent_s)
    pltpu.sync_copy(sent_e_h, sent_e)

    for i in range(P // 16):
        sl = pl.ds(i * 16, 16)
        s_v = st_ref[sl]
        e_v = en_ref[sl]
        fs_idx[sl] = base2 + 2 * jnp.maximum(s_v - 1, 0)
        fe_idx[sl] = base2 + 2 * e_v
        bx_idx[sl] = base2 + 2 * jnp.minimum(e_v + 1, S - 1) + 1
        bs_idx[sl] = base2 + 2 * s_v + 1
        ms_ref[sl] = (1 - jnp.minimum(s_v, 1)).astype(jnp.float32)
        me_ref[sl] = (1 - jnp.minimum((S - 1) - e_v, 1)).astype(jnp.float32)
        w_v = e_v - s_v
        one = jnp.int32(1)
        zero = jnp.int32(0)
        bkt = (jnp.minimum(w_v, 4)
               + jnp.minimum(jnp.maximum(w_v - 4, zero), one)
               + jnp.minimum(jnp.maximum(w_v - 7, zero), one)
               + jnp.minimum(jnp.maximum(w_v - 15, zero), one)
               + jnp.minimum(jnp.maximum(w_v - 31, zero), one)
               + jnp.minimum(jnp.maximum(w_v - 63, zero), one))
        wb_idx[i // 8, pl.ds((i % 8) * 16, 16)] = bkt

    wcp0 = pltpu.async_copy(wemb_h.at[wb_idx.at[0]],
                            wemb_buf.at[pl.ds(0, 128)], sem_w)
    wcp1 = pltpu.async_copy(wemb_h.at[wb_idx.at[1]],
                            wemb_buf.at[pl.ds(128, 128)], sem_w)

    def chunk(c, carry):
        off = pl.multiple_of(c * C, C)
        sl = pl.ds(off, C)
        cp0 = pltpu.async_copy(seq2.at[fs_idx[sl]], fs_buf, sem_g)
        cp1 = pltpu.async_copy(seq2.at[fe_idx[sl]], fe_buf, sem_g)
        cp2 = pltpu.async_copy(seq2.at[bx_idx[sl]], bx_buf, sem_g)
        cp3 = pltpu.async_copy(seq2.at[bs_idx[sl]], bs_buf, sem_g)
        cp0.wait()
        cp1.wait()
        cp2.wait()
        cp3.wait()

        ms_chunk = ms_ref[sl]
        me_chunk = me_ref[sl]

        def span_body(i, carry2):
            gvec = jnp.full((16,), i, jnp.int32)
            ms = ms_chunk.at[gvec].get(mode="promise_in_bounds")
            me = me_chunk.at[gvec].get(mode="promise_in_bounds")

            def dim_body(j, carry3):
                jo = pl.multiple_of(j * 16, 16)
                dsl = pl.ds(jo, 16)
                fe_v = fe_buf[i, dsl]
                fs_v = fs_buf[i, dsl]
                bx_v = bx_buf[i, dsl]
                bs_v = bs_buf[i, dsl]
                fe_buf[i, dsl] = fe_v - fs_v * (1.0 - ms) - sent_s[dsl] * ms
                bs_buf[i, dsl] = bx_v * (1.0 - me) + sent_e[dsl] * me - bs_v
                return carry3

            return lax.fori_loop(0, HALF // 16, dim_body, carry2)

        lax.fori_loop(0, C, span_body, 0)

        pltpu.sync_copy(fe_buf, out.at[pl.ds(r0 + off, C), pl.ds(0, HALF)])
        pltpu.sync_copy(bs_buf, out.at[pl.ds(r0 + off, C), pl.ds(HALF, HALF)])
        return carry

    lax.fori_loop(0, NCHUNK, chunk, 0)

    wcp0.wait()
    wcp1.wait()
    pltpu.sync_copy(wemb_buf, out.at[pl.ds(r0, P), pl.ds(2 * HALF, 128)])


_sc_call = functools.partial(
    pl.kernel,
    mesh=plsc.VectorSubcoreMesh(core_axis_name="c", subcore_axis_name="s"),
    out_type=jax.ShapeDtypeStruct((B * NUM_SPANS, OUT_D_PAD), jnp.float32),
    scratch_types=[
        pltpu.VMEM((C, HALF), jnp.float32),
        pltpu.VMEM((C, HALF), jnp.float32),
        pltpu.VMEM((C, HALF), jnp.float32),
        pltpu.VMEM((C, HALF), jnp.float32),
        pltpu.VMEM((P, 128), jnp.float32),
        pltpu.VMEM((HALF,), jnp.float32),
        pltpu.VMEM((HALF,), jnp.float32),
        pltpu.VMEM((P,), jnp.int32),
        pltpu.VMEM((P,), jnp.int32),
        pltpu.VMEM((P,), jnp.int32),
        pltpu.VMEM((P,), jnp.int32),
        pltpu.VMEM((P,), jnp.int32),
        pltpu.VMEM((P,), jnp.int32),
        pltpu.VMEM((P,), jnp.float32),
        pltpu.VMEM((P,), jnp.float32),
        pltpu.VMEM((2, 128), jnp.int32),
        pltpu.SemaphoreType.DMA,
        pltpu.SemaphoreType.DMA,
    ],
)(_sc_body)


def kernel(sequence_tensor, span_indices, start_sentinel, end_sentinel,
           width_embedding):
    seq2 = sequence_tensor.reshape(B * S * 2, HALF)
    starts = span_indices[..., 0].reshape(-1).astype(jnp.int32)
    ends = span_indices[..., 1].reshape(-1).astype(jnp.int32)
    wemb_p = jnp.zeros((width_embedding.shape[0], 128),
                       width_embedding.dtype).at[:, :WIDTH_DIM].set(width_embedding)
    out = _sc_call(seq2, starts, ends,
                   start_sentinel.reshape(HALF),
                   end_sentinel.reshape(HALF),
                   wemb_p)
    return out[:, :OUT_D].reshape(B, NUM_SPANS, OUT_D)


# speedup vs baseline: 1.0958x; 1.0958x over previous
import functools

import jax
import jax.numpy as jnp
from jax import lax
from jax.experimental import pallas as pl
from jax.experimental.pallas import tpu as pltpu
from jax.experimental.pallas import tpu_sc as plsc

B, S, D = 4, 2048, 2048
HALF = D // 2
NUM_SPANS = 2048
WIDTH_DIM = 64
OUT_D = 2 * HALF + WIDTH_DIM
OUT_D_PAD = 2 * HALF + 128

NW = 32
P = (B * NUM_SPANS) // NW
C = 16
NCHUNK = P // C
WPB = NW // B


def _sc_body(seq2, starts, ends, sent_s_h, sent_e_h, wemb_h, out,
             fs_buf, fe_buf, bx_buf, bs_buf, wemb_buf,
             sent_s, sent_e, st_ref, en_ref,
             fs_idx, fe_idx, bx_idx, bs_idx, ms_ref, me_ref, wb_idx,
             sem_g, sem_w):
    c_id = lax.axis_index("c")
    s_id = lax.axis_index("s")
    wid = s_id * 2 + c_id
    r0 = pl.multiple_of(wid * P, P)
    base2 = jnp.full((16,), (wid // WPB) * (2 * S), jnp.int32)

    pltpu.sync_copy(starts.at[pl.ds(r0, P)], st_ref)
    pltpu.sync_copy(ends.at[pl.ds(r0, P)], en_ref)
    pltpu.sync_copy(sent_s_h, sent_s)
    pltpu.sync_copy(sent_e_h, sent_e)

    for i in range(P // 16):
        sl = pl.ds(i * 16, 16)
        s_v = st_ref[sl]
        e_v = en_ref[sl]
        fs_idx[sl] = base2 + 2 * jnp.maximum(s_v - 1, 0)
        fe_idx[sl] = base2 + 2 * e_v
        bx_idx[sl] = base2 + 2 * jnp.minimum(e_v + 1, S - 1) + 1
        bs_idx[sl] = base2 + 2 * s_v + 1
        ms_ref[sl] = (1 - jnp.minimum(s_v, 1)).astype(jnp.float32)
        me_ref[sl] = (1 - jnp.minimum((S - 1) - e_v, 1)).astype(jnp.float32)
        w_v = e_v - s_v
        one = jnp.int32(1)
        zero = jnp.int32(0)
        bkt = (jnp.minimum(w_v, 4)
               + jnp.minimum(jnp.maximum(w_v - 4, zero), one)
               + jnp.minimum(jnp.maximum(w_v - 7, zero), one)
               + jnp.minimum(jnp.maximum(w_v - 15, zero), one)
               + jnp.minimum(jnp.maximum(w_v - 31, zero), one)
               + jnp.minimum(jnp.maximum(w_v - 63, zero), one))
        wb_idx[i // 8, pl.ds((i % 8) * 16, 16)] = bkt

    wcp0 = pltpu.async_copy(wemb_h.at[wb_idx.at[0]],
                            wemb_buf.at[pl.ds(0, 128)], sem_w)
    wcp1 = pltpu.async_copy(wemb_h.at[wb_idx.at[1]],
                            wemb_buf.at[pl.ds(128, 128)], sem_w)

    def chunk(c, carry):
        off = pl.multiple_of(c * C, C)
        sl = pl.ds(off, C)
        cp0 = pltpu.async_copy(seq2.at[fs_idx[sl]], fs_buf, sem_g)
        cp1 = pltpu.async_copy(seq2.at[fe_idx[sl]], fe_buf, sem_g)
        cp2 = pltpu.async_copy(seq2.at[bx_idx[sl]], bx_buf, sem_g)
        cp3 = pltpu.async_copy(seq2.at[bs_idx[sl]], bs_buf, sem_g)
        cp0.wait()
        cp1.wait()
        cp2.wait()
        cp3.wait()

        ms_chunk = ms_ref[sl]
        me_chunk = me_ref[sl]

        def span_body(i, carry2):
            gvec = jnp.full((16,), i, jnp.int32)
            ms = ms_chunk.at[gvec].get(mode="promise_in_bounds")
            me = me_chunk.at[gvec].get(mode="promise_in_bounds")
            norm = (ms + me) + 0.0 * lax.iota(jnp.int32, 16).astype(jnp.float32)
            has_sent = norm[0] > 0.0

            def slow_path():
                def dim_body(j, carry3):
                    jo = pl.multiple_of(j * 16, 16)
                    dsl = pl.ds(jo, 16)
                    fe_v = fe_buf[i, dsl]
                    fs_v = fs_buf[i, dsl]
                    bx_v = bx_buf[i, dsl]
                    bs_v = bs_buf[i, dsl]
                    fe_buf[i, dsl] = fe_v - fs_v * (1.0 - ms) - sent_s[dsl] * ms
                    bs_buf[i, dsl] = bx_v * (1.0 - me) + sent_e[dsl] * me - bs_v
                    return carry3

                lax.fori_loop(0, HALF // 16, dim_body, 0)

            def fast_path():
                def dim_body(j, carry3):
                    for u in range(4):
                        jo = pl.multiple_of(j * 64 + u * 16, 16)
                        dsl = pl.ds(jo, 16)
                        fe_buf[i, dsl] = fe_buf[i, dsl] - fs_buf[i, dsl]
                        bs_buf[i, dsl] = bx_buf[i, dsl] - bs_buf[i, dsl]
                    return carry3

                lax.fori_loop(0, HALF // 64, dim_body, 0)

            lax.cond(has_sent, slow_path, fast_path)
            return carry2

        lax.fori_loop(0, C, span_body, 0)

        pltpu.sync_copy(fe_buf, out.at[pl.ds(r0 + off, C), pl.ds(0, HALF)])
        pltpu.sync_copy(bs_buf, out.at[pl.ds(r0 + off, C), pl.ds(HALF, HALF)])
        return carry

    lax.fori_loop(0, NCHUNK, chunk, 0)

    wcp0.wait()
    wcp1.wait()
    pltpu.sync_copy(wemb_buf, out.at[pl.ds(r0, P), pl.ds(2 * HALF, 128)])


_sc_call = functools.partial(
    pl.kernel,
    mesh=plsc.VectorSubcoreMesh(core_axis_name="c", subcore_axis_name="s"),
    out_type=jax.ShapeDtypeStruct((B * NUM_SPANS, OUT_D_PAD), jnp.float32),
    scratch_types=[
        pltpu.VMEM((C, HALF), jnp.float32),
        pltpu.VMEM((C, HALF), jnp.float32),
        pltpu.VMEM((C, HALF), jnp.float32),
        pltpu.VMEM((C, HALF), jnp.float32),
        pltpu.VMEM((P, 128), jnp.float32),
        pltpu.VMEM((HALF,), jnp.float32),
        pltpu.VMEM((HALF,), jnp.float32),
        pltpu.VMEM((P,), jnp.int32),
        pltpu.VMEM((P,), jnp.int32),
        pltpu.VMEM((P,), jnp.int32),
        pltpu.VMEM((P,), jnp.int32),
        pltpu.VMEM((P,), jnp.int32),
        pltpu.VMEM((P,), jnp.int32),
        pltpu.VMEM((P,), jnp.float32),
        pltpu.VMEM((P,), jnp.float32),
        pltpu.VMEM((2, 128), jnp.int32),
        pltpu.SemaphoreType.DMA,
        pltpu.SemaphoreType.DMA,
    ],
)(_sc_body)


def kernel(sequence_tensor, span_indices, start_sentinel, end_sentinel,
           width_embedding):
    seq2 = sequence_tensor.reshape(B * S * 2, HALF)
    starts = span_indices[..., 0].reshape(-1).astype(jnp.int32)
    ends = span_indices[..., 1].reshape(-1).astype(jnp.int32)
    wemb_p = jnp.zeros((width_embedding.shape[0], 128),
                       width_embedding.dtype).at[:, :WIDTH_DIM].set(width_embedding)
    out = _sc_call(seq2, starts, ends,
                   start_sentinel.reshape(HALF),
                   end_sentinel.reshape(HALF),
                   wemb_p)
    return out[:, :OUT_D].reshape(B, NUM_SPANS, OUT_D)


# speedup vs baseline: 1.3847x; 1.2637x over previous
import functools

import jax
import jax.numpy as jnp
from jax import lax
from jax.experimental import pallas as pl
from jax.experimental.pallas import tpu as pltpu
from jax.experimental.pallas import tpu_sc as plsc

B, S, D = 4, 2048, 2048
HALF = D // 2
NUM_SPANS = 2048
WIDTH_DIM = 64
OUT_D = 2 * HALF + WIDTH_DIM
OUT_D_PAD = 2 * HALF + 128

NW = 32
P = (B * NUM_SPANS) // NW
C = 8
NCHUNK = P // C
WPB = NW // B


def _sc_body(seq2, starts, ends, sent_s_h, sent_e_h, wemb_h, out,
             fs0, fe0, bx0, bs0, fs1, fe1, bx1, bs1,
             wf0, wb0, wf1, wb1, wemb_buf,
             sent_s, sent_e, st_ref, en_ref,
             fs_idx, fe_idx, bx_idx, bs_idx, ms_ref, me_ref, wb_idx,
             sem_g0, sem_g1, sem_wr0, sem_wr1, sem_w):
    c_id = lax.axis_index("c")
    s_id = lax.axis_index("s")
    wid = s_id * 2 + c_id
    r0 = pl.multiple_of(wid * P, P)
    base2 = jnp.full((16,), (wid // WPB) * (2 * S), jnp.int32)

    gsets = ((fs0, fe0, bx0, bs0, sem_g0), (fs1, fe1, bx1, bs1, sem_g1))
    wsets = ((wf0, wb0, sem_wr0), (wf1, wb1, sem_wr1))

    pltpu.sync_copy(starts.at[pl.ds(r0, P)], st_ref)
    pltpu.sync_copy(ends.at[pl.ds(r0, P)], en_ref)
    pltpu.sync_copy(sent_s_h, sent_s)
    pltpu.sync_copy(sent_e_h, sent_e)

    for i in range(P // 16):
        sl = pl.ds(i * 16, 16)
        s_v = st_ref[sl]
        e_v = en_ref[sl]
        fs_idx[sl] = base2 + 2 * jnp.maximum(s_v - 1, 0)
        fe_idx[sl] = base2 + 2 * e_v
        bx_idx[sl] = base2 + 2 * jnp.minimum(e_v + 1, S - 1) + 1
        bs_idx[sl] = base2 + 2 * s_v + 1
        ms_ref[sl] = (1 - jnp.minimum(s_v, 1)).astype(jnp.float32)
        me_ref[sl] = (1 - jnp.minimum((S - 1) - e_v, 1)).astype(jnp.float32)
        w_v = e_v - s_v
        one = jnp.int32(1)
        zero = jnp.int32(0)
        bkt = (jnp.minimum(w_v, 4)
               + jnp.minimum(jnp.maximum(w_v - 4, zero), one)
               + jnp.minimum(jnp.maximum(w_v - 7, zero), one)
               + jnp.minimum(jnp.maximum(w_v - 15, zero), one)
               + jnp.minimum(jnp.maximum(w_v - 31, zero), one)
               + jnp.minimum(jnp.maximum(w_v - 63, zero), one))
        wb_idx[i // 8, pl.ds((i % 8) * 16, 16)] = bkt

    def fire_gathers(c, par):
        fsb, feb, bxb, bsb, sem = gsets[par]
        off = pl.multiple_of(c * C, C)
        isl = pl.ds(off, C)
        pltpu.async_copy(seq2.at[fs_idx.at[isl]], fsb, sem)
        pltpu.async_copy(seq2.at[fe_idx.at[isl]], feb, sem)
        pltpu.async_copy(seq2.at[bx_idx.at[isl]], bxb, sem)
        pltpu.async_copy(seq2.at[bs_idx.at[isl]], bsb, sem)

    def wait_gathers(c, par):
        fsb, feb, bxb, bsb, sem = gsets[par]
        off = pl.multiple_of(c * C, C)
        isl = pl.ds(off, C)
        pltpu.make_async_copy(seq2.at[fs_idx.at[isl]], fsb, sem).wait()
        pltpu.make_async_copy(seq2.at[fe_idx.at[isl]], feb, sem).wait()
        pltpu.make_async_copy(seq2.at[bx_idx.at[isl]], bxb, sem).wait()
        pltpu.make_async_copy(seq2.at[bs_idx.at[isl]], bsb, sem).wait()

    def out_slabs(c, par):
        wfb, wbb, sem = wsets[par]
        rows = pl.ds(r0 + pl.multiple_of(c * C, C), C)
        return ((wfb, out.at[rows, pl.ds(0, HALF)], sem),
                (wbb, out.at[rows, pl.ds(HALF, HALF)], sem))

    def fire_writes(c, par):
        for src, dst, sem in out_slabs(c, par):
            pltpu.async_copy(src, dst, sem)

    def wait_writes(c, par):
        for src, dst, sem in out_slabs(c, par):
            pltpu.make_async_copy(src, dst, sem).wait()

    def compute_chunk(c, par):
        fsb, feb, bxb, bsb, _ = gsets[par]
        wfb, wbb, _ = wsets[par]
        off = pl.multiple_of(c * C, C)
        ms_chunk = ms_ref[pl.ds(off, 16)]
        me_chunk = me_ref[pl.ds(off, 16)]

        def span_body(i, carry2):
            gvec = jnp.full((16,), i, jnp.int32)
            ms = ms_chunk.at[gvec].get(mode="promise_in_bounds")
            me = me_chunk.at[gvec].get(mode="promise_in_bounds")
            norm = (ms + me) + 0.0 * lax.iota(jnp.int32, 16).astype(jnp.float32)
            has_sent = norm[0] > 0.0

            def slow_path():
                def dim_body(j, carry3):
                    jo = pl.multiple_of(j * 16, 16)
                    dsl = pl.ds(jo, 16)
                    wfb[i, dsl] = (feb[i, dsl] - fsb[i, dsl] * (1.0 - ms)
                                   - sent_s[dsl] * ms)
                    wbb[i, dsl] = (bxb[i, dsl] * (1.0 - me)
                                   + sent_e[dsl] * me - bsb[i, dsl])
                    return carry3

                lax.fori_loop(0, HALF // 16, dim_body, 0)

            def fast_path():
                def dim_body(j, carry3):
                    for u in range(4):
                        jo = pl.multiple_of(j * 64 + u * 16, 16)
                        dsl = pl.ds(jo, 16)
                        wfb[i, dsl] = feb[i, dsl] - fsb[i, dsl]
                        wbb[i, dsl] = bxb[i, dsl] - bsb[i, dsl]
                    return carry3

                lax.fori_loop(0, HALF // 64, dim_body, 0)

            lax.cond(has_sent, slow_path, fast_path)
            return carry2

        lax.fori_loop(0, C, span_body, 0)

    fire_gathers(0, 0)

    def pair_body(g, carry):
        for par in (0, 1):
            c = g * 2 + par

            @pl.when(c + 1 < NCHUNK)
            def _():
                fire_gathers(c + 1, 1 - par)

            wait_gathers(c, par)

            @pl.when(c >= 2)
            def _():
                wait_writes(c - 2, par)

            compute_chunk(c, par)
            fire_writes(c, par)
        return carry

    lax.fori_loop(0, NCHUNK // 2, pair_body, 0)
    wait_writes(NCHUNK - 2, 0)
    wait_writes(NCHUNK - 1, 1)

    for h in range(2):
        hoff = pl.multiple_of(h * 128, 128)
        cp = pltpu.async_copy(wemb_h.at[wb_idx.at[h]], wemb_buf, sem_w)
        cp.wait()
        pltpu.sync_copy(wemb_buf,
                        out.at[pl.ds(r0 + hoff, 128), pl.ds(2 * HALF, 128)])


_sc_call = functools.partial(
    pl.kernel,
    mesh=plsc.VectorSubcoreMesh(core_axis_name="c", subcore_axis_name="s"),
    out_type=jax.ShapeDtypeStruct((B * NUM_SPANS, OUT_D_PAD), jnp.float32),
    scratch_types=[
        pltpu.VMEM((C, HALF), jnp.float32),
        pltpu.VMEM((C, HALF), jnp.float32),
        pltpu.VMEM((C, HALF), jnp.float32),
        pltpu.VMEM((C, HALF), jnp.float32),
        pltpu.VMEM((C, HALF), jnp.float32),
        pltpu.VMEM((C, HALF), jnp.float32),
        pltpu.VMEM((C, HALF), jnp.float32),
        pltpu.VMEM((C, HALF), jnp.float32),
        pltpu.VMEM((C, HALF), jnp.float32),
        pltpu.VMEM((C, HALF), jnp.float32),
        pltpu.VMEM((C, HALF), jnp.float32),
        pltpu.VMEM((C, HALF), jnp.float32),
        pltpu.VMEM((128, 128), jnp.float32),
        pltpu.VMEM((HALF,), jnp.float32),
        pltpu.VMEM((HALF,), jnp.float32),
        pltpu.VMEM((P,), jnp.int32),
        pltpu.VMEM((P,), jnp.int32),
        pltpu.VMEM((P,), jnp.int32),
        pltpu.VMEM((P,), jnp.int32),
        pltpu.VMEM((P,), jnp.int32),
        pltpu.VMEM((P,), jnp.int32),
        pltpu.VMEM((P + 16,), jnp.float32),
        pltpu.VMEM((P + 16,), jnp.float32),
        pltpu.VMEM((2, 128), jnp.int32),
        pltpu.SemaphoreType.DMA,
        pltpu.SemaphoreType.DMA,
        pltpu.SemaphoreType.DMA,
        pltpu.SemaphoreType.DMA,
        pltpu.SemaphoreType.DMA,
    ],
)(_sc_body)


def kernel(sequence_tensor, span_indices, start_sentinel, end_sentinel,
           width_embedding):
    seq2 = sequence_tensor.reshape(B * S * 2, HALF)
    starts = span_indices[..., 0].reshape(-1).astype(jnp.int32)
    ends = span_indices[..., 1].reshape(-1).astype(jnp.int32)
    wemb_p = jnp.zeros((width_embedding.shape[0], 128),
                       width_embedding.dtype).at[:, :WIDTH_DIM].set(width_embedding)
    out = _sc_call(seq2, starts, ends,
                   start_sentinel.reshape(HALF),
                   end_sentinel.reshape(HALF),
                   wemb_p)
    return out[:, :OUT_D].reshape(B, NUM_SPANS, OUT_D)


# speedup vs baseline: 1.5151x; 1.0942x over previous
import functools

import jax
import jax.numpy as jnp
from jax import lax
from jax.experimental import pallas as pl
from jax.experimental.pallas import tpu as pltpu
from jax.experimental.pallas import tpu_sc as plsc

B, S, D = 4, 2048, 2048
HALF = D // 2
NUM_SPANS = 2048
WIDTH_DIM = 64
OUT_D = 2 * HALF + WIDTH_DIM

NW = 32
P = (B * NUM_SPANS) // NW
C = 8
NCHUNK = P // C
WPB = NW // B


def _sc_body(seq2, starts, ends, sent_s_h, sent_e_h, wemb_h, out,
             fs0, fe0, bx0, bs0, fs1, fe1, bx1, bs1,
             wc0, wc1, wemb_buf,
             sent_s, sent_e, st_ref, en_ref,
             fs_idx, fe_idx, bx_idx, bs_idx, ms_ref, me_ref, wb_idx,
             sem_g0, sem_g1, sem_wr0, sem_wr1, sem_w):
    c_id = lax.axis_index("c")
    s_id = lax.axis_index("s")
    wid = s_id * 2 + c_id
    r0 = pl.multiple_of(wid * P, P)
    base2 = jnp.full((16,), (wid // WPB) * (2 * S), jnp.int32)

    gsets = ((fs0, fe0, bx0, bs0, sem_g0), (fs1, fe1, bx1, bs1, sem_g1))
    wsets = ((wc0, sem_wr0), (wc1, sem_wr1))

    pltpu.sync_copy(starts.at[pl.ds(r0, P)], st_ref)
    pltpu.sync_copy(ends.at[pl.ds(r0, P)], en_ref)
    pltpu.sync_copy(sent_s_h, sent_s)
    pltpu.sync_copy(sent_e_h, sent_e)

    for i in range(P // 16):
        sl = pl.ds(i * 16, 16)
        s_v = st_ref[sl]
        e_v = en_ref[sl]
        fs_idx[sl] = base2 + 2 * jnp.maximum(s_v - 1, 0)
        fe_idx[sl] = base2 + 2 * e_v
        bx_idx[sl] = base2 + 2 * jnp.minimum(e_v + 1, S - 1) + 1
        bs_idx[sl] = base2 + 2 * s_v + 1
        ms_ref[sl] = (1 - jnp.minimum(s_v, 1)).astype(jnp.float32)
        me_ref[sl] = (1 - jnp.minimum((S - 1) - e_v, 1)).astype(jnp.float32)
        w_v = e_v - s_v
        one = jnp.int32(1)
        zero = jnp.int32(0)
        bkt = (jnp.minimum(w_v, 4)
               + jnp.minimum(jnp.maximum(w_v - 4, zero), one)
               + jnp.minimum(jnp.maximum(w_v - 7, zero), one)
               + jnp.minimum(jnp.maximum(w_v - 15, zero), one)
               + jnp.minimum(jnp.maximum(w_v - 31, zero), one)
               + jnp.minimum(jnp.maximum(w_v - 63, zero), one))
        wb_idx[i // 8, pl.ds((i % 8) * 16, 16)] = bkt

    def fire_gathers(c, par):
        fsb, feb, bxb, bsb, sem = gsets[par]
        isl = pl.ds(pl.multiple_of(c * C, C), C)
        pltpu.async_copy(seq2.at[fs_idx.at[isl]], fsb, sem)
        pltpu.async_copy(seq2.at[fe_idx.at[isl]], feb, sem)
        pltpu.async_copy(seq2.at[bx_idx.at[isl]], bxb, sem)
        pltpu.async_copy(seq2.at[bs_idx.at[isl]], bsb, sem)

    def wait_gathers(c, par):
        fsb, feb, bxb, bsb, sem = gsets[par]
        isl = pl.ds(pl.multiple_of(c * C, C), C)
        pltpu.make_async_copy(seq2.at[fs_idx.at[isl]], fsb, sem).wait()
        pltpu.make_async_copy(seq2.at[fe_idx.at[isl]], feb, sem).wait()
        pltpu.make_async_copy(seq2.at[bx_idx.at[isl]], bxb, sem).wait()
        pltpu.make_async_copy(seq2.at[bs_idx.at[isl]], bsb, sem).wait()

    def out_slab(c, par):
        wcb, sem = wsets[par]
        rows = pl.ds(r0 + pl.multiple_of(c * C, C), C)
        return wcb, out.at[rows, pl.ds(0, 2 * HALF)], sem

    def compute_chunk(c, par):
        fsb, feb, bxb, bsb, _ = gsets[par]
        wcb, _ = wsets[par]
        off = pl.multiple_of(c * C, C)
        ms_chunk = ms_ref[pl.ds(off, 16)]
        me_chunk = me_ref[pl.ds(off, 16)]

        def span_body(i, carry2):
            gvec = jnp.full((16,), i, jnp.int32)
            ms = ms_chunk.at[gvec].get(mode="promise_in_bounds")
            me = me_chunk.at[gvec].get(mode="promise_in_bounds")
            norm = (ms + me) + 0.0 * lax.iota(jnp.int32, 16).astype(jnp.float32)
            has_sent = norm[0] > 0.0

            def slow_path():
                def dim_body(j, carry3):
                    jo = pl.multiple_of(j * 16, 16)
                    dsl = pl.ds(jo, 16)
                    bsl = pl.ds(HALF + jo, 16)
                    wcb[i, dsl] = (feb[i, dsl] - fsb[i, dsl] * (1.0 - ms)
                                   - sent_s[dsl] * ms)
                    wcb[i, bsl] = (bxb[i, dsl] * (1.0 - me)
                                   + sent_e[dsl] * me - bsb[i, dsl])
                    return carry3

                lax.fori_loop(0, HALF // 16, dim_body, 0)

            def fast_path():
                for j in range(HALF // 16):
                    dsl = pl.ds(j * 16, 16)
                    bsl = pl.ds(HALF + j * 16, 16)
                    wcb[i, dsl] = feb[i, dsl] - fsb[i, dsl]
                    wcb[i, bsl] = bxb[i, dsl] - bsb[i, dsl]

            lax.cond(has_sent, slow_path, fast_path)
            return carry2

        lax.fori_loop(0, C, span_body, 0)

    fire_gathers(0, 0)

    def pair_body(g, carry):
        for par in (0, 1):
            c = g * 2 + par

            @pl.when(c + 1 < NCHUNK)
            def _():
                fire_gathers(c + 1, 1 - par)

            wait_gathers(c, par)

            @pl.when(c >= 2)
            def _():
                src, dst, sem = out_slab(c - 2, par)
                pltpu.make_async_copy(src, dst, sem).wait()

            compute_chunk(c, par)
            src, dst, sem = out_slab(c, par)
            pltpu.async_copy(src, dst, sem)
        return carry

    lax.fori_loop(0, NCHUNK // 2, pair_body, 0)
    for c, par in ((NCHUNK - 2, 0), (NCHUNK - 1, 1)):
        src, dst, sem = out_slab(c, par)
        pltpu.make_async_copy(src, dst, sem).wait()

    for h in range(2):
        hbase = pl.multiple_of(r0 + h * 128, 128)
        cp = pltpu.async_copy(wemb_h.at[wb_idx.at[h]], wemb_buf, sem_w)
        cp.wait()

        def wrow(r, carry):
            src = wemb_buf.at[r, pl.ds(0, WIDTH_DIM)]
            dst = out.at[hbase + r, pl.ds(2 * HALF, WIDTH_DIM)]
            pltpu.async_copy(src, dst, sem_w)
            return carry

        lax.fori_loop(0, 128, wrow, 0)

        def wrow_wait(r, carry):
            src = wemb_buf.at[r, pl.ds(0, WIDTH_DIM)]
            dst = out.at[hbase + r, pl.ds(2 * HALF, WIDTH_DIM)]
            pltpu.make_async_copy(src, dst, sem_w).wait()
            return carry

        lax.fori_loop(0, 128, wrow_wait, 0)


_sc_call = functools.partial(
    pl.kernel,
    mesh=plsc.VectorSubcoreMesh(core_axis_name="c", subcore_axis_name="s"),
    out_type=jax.ShapeDtypeStruct((B * NUM_SPANS, OUT_D), jnp.float32),
    scratch_types=[
        pltpu.VMEM((C, HALF), jnp.float32),
        pltpu.VMEM((C, HALF), jnp.float32),
        pltpu.VMEM((C, HALF), jnp.float32),
        pltpu.VMEM((C, HALF), jnp.float32),
        pltpu.VMEM((C, HALF), jnp.float32),
        pltpu.VMEM((C, HALF), jnp.float32),
        pltpu.VMEM((C, HALF), jnp.float32),
        pltpu.VMEM((C, HALF), jnp.float32),
        pltpu.VMEM((C, 2 * HALF), jnp.float32),
        pltpu.VMEM((C, 2 * HALF), jnp.float32),
        pltpu.VMEM((128, 128), jnp.float32),
        pltpu.VMEM((HALF,), jnp.float32),
        pltpu.VMEM((HALF,), jnp.float32),
        pltpu.VMEM((P,), jnp.int32),
        pltpu.VMEM((P,), jnp.int32),
        pltpu.VMEM((P,), jnp.int32),
        pltpu.VMEM((P,), jnp.int32),
        pltpu.VMEM((P,), jnp.int32),
        pltpu.VMEM((P,), jnp.int32),
        pltpu.VMEM((P + 16,), jnp.float32),
        pltpu.VMEM((P + 16,), jnp.float32),
        pltpu.VMEM((2, 128), jnp.int32),
        pltpu.SemaphoreType.DMA,
        pltpu.SemaphoreType.DMA,
        pltpu.SemaphoreType.DMA,
        pltpu.SemaphoreType.DMA,
        pltpu.SemaphoreType.DMA,
    ],
)(_sc_body)


def kernel(sequence_tensor, span_indices, start_sentinel, end_sentinel,
           width_embedding):
    seq2 = sequence_tensor.reshape(B * S * 2, HALF)
    starts = span_indices[..., 0].reshape(-1).astype(jnp.int32)
    ends = span_indices[..., 1].reshape(-1).astype(jnp.int32)
    wemb_p = jnp.zeros((width_embedding.shape[0], 128),
                       width_embedding.dtype).at[:, :WIDTH_DIM].set(width_embedding)
    out = _sc_call(seq2, starts, ends,
                   start_sentinel.reshape(HALF),
                   end_sentinel.reshape(HALF),
                   wemb_p)
    return out.reshape(B, NUM_SPANS, OUT_D)


# speedup vs baseline: 1.8407x; 1.2149x over previous
import functools

import jax
import jax.numpy as jnp
from jax import lax
from jax.experimental import pallas as pl
from jax.experimental.pallas import tpu as pltpu
from jax.experimental.pallas import tpu_sc as plsc

B, S, D = 4, 2048, 2048
HALF = D // 2
NUM_SPANS = 2048
WIDTH_DIM = 64
OUT_D = 2 * HALF + WIDTH_DIM

NW = 32
P = (B * NUM_SPANS) // NW
C = 8
NCHUNK = P // C
WPB = NW // B


def _sc_body(seq2, starts, ends, sent_s_h, sent_e_h, wemb_h, out,
             fs0, fe0, bx0, bs0, fs1, fe1, bx1, bs1,
             wc0, wc1, wemb_buf,
             sent_s, sent_e, st_ref, en_ref,
             fs_idx, fe_idx, bx_idx, bs_idx, ms_ref, me_ref, wb_idx,
             sem_g0, sem_g1, sem_wr0, sem_wr1, sem_w):
    c_id = lax.axis_index("c")
    s_id = lax.axis_index("s")
    wid = s_id * 2 + c_id
    r0 = pl.multiple_of(wid * P, P)
    baseS = jnp.full((16,), (wid // WPB) * S, jnp.int32)

    gsets = ((fs0, fe0, bx0, bs0, sem_g0), (fs1, fe1, bx1, bs1, sem_g1))
    wsets = ((wc0, sem_wr0), (wc1, sem_wr1))

    pltpu.sync_copy(starts.at[pl.ds(r0, P)], st_ref)
    pltpu.sync_copy(ends.at[pl.ds(r0, P)], en_ref)
    pltpu.sync_copy(sent_s_h, sent_s)
    pltpu.sync_copy(sent_e_h, sent_e)

    for i in range(P // 16):
        sl = pl.ds(i * 16, 16)
        s_v = st_ref[sl]
        e_v = en_ref[sl]
        fs_idx[sl] = baseS + jnp.maximum(s_v - 1, 0)
        fe_idx[sl] = baseS + e_v
        bx_idx[sl] = baseS + jnp.minimum(e_v + 1, S - 1)
        bs_idx[sl] = baseS + s_v
        ms_ref[sl] = (1 - jnp.minimum(s_v, 1)).astype(jnp.float32)
        me_ref[sl] = (1 - jnp.minimum((S - 1) - e_v, 1)).astype(jnp.float32)
        w_v = e_v - s_v
        one = jnp.int32(1)
        zero = jnp.int32(0)
        bkt = (jnp.minimum(w_v, 4)
               + jnp.minimum(jnp.maximum(w_v - 4, zero), one)
               + jnp.minimum(jnp.maximum(w_v - 7, zero), one)
               + jnp.minimum(jnp.maximum(w_v - 15, zero), one)
               + jnp.minimum(jnp.maximum(w_v - 31, zero), one)
               + jnp.minimum(jnp.maximum(w_v - 63, zero), one))
        wb_idx[i // 8, pl.ds((i % 8) * 16, 16)] = bkt

    def gather_pairs(c, par):
        fsb, feb, bxb, bsb, sem = gsets[par]
        isl = pl.ds(pl.multiple_of(c * C, C), C)
        fsl = pl.ds(0, HALF)
        bsl = pl.ds(HALF, HALF)
        return ((seq2.at[fs_idx.at[isl], fsl], fsb, sem),
                (seq2.at[fe_idx.at[isl], fsl], feb, sem),
                (seq2.at[bx_idx.at[isl], bsl], bxb, sem),
                (seq2.at[bs_idx.at[isl], bsl], bsb, sem))

    def fire_gathers(c, par):
        for src, dst, sem in gather_pairs(c, par):
            pltpu.async_copy(src, dst, sem)

    def wait_gathers(c, par):
        for src, dst, sem in gather_pairs(c, par):
            pltpu.make_async_copy(src, dst, sem).wait()

    def out_slab(c, par):
        wcb, sem = wsets[par]
        rows = pl.ds(r0 + pl.multiple_of(c * C, C), C)
        return wcb, out.at[rows, pl.ds(0, 2 * HALF)], sem

    def compute_chunk(c, par):
        fsb, feb, bxb, bsb, _ = gsets[par]
        wcb, _ = wsets[par]
        off = pl.multiple_of(c * C, C)
        ms_chunk = ms_ref[pl.ds(off, 16)]
        me_chunk = me_ref[pl.ds(off, 16)]

        def span_body(i, carry2):
            gvec = jnp.full((16,), i, jnp.int32)
            ms = ms_chunk.at[gvec].get(mode="promise_in_bounds")
            me = me_chunk.at[gvec].get(mode="promise_in_bounds")
            norm = (ms + me) + 0.0 * lax.iota(jnp.int32, 16).astype(jnp.float32)
            has_sent = norm[0] > 0.0

            def slow_path():
                def dim_body(j, carry3):
                    jo = pl.multiple_of(j * 16, 16)
                    dsl = pl.ds(jo, 16)
                    bsl = pl.ds(HALF + jo, 16)
                    wcb[i, dsl] = (feb[i, dsl] - fsb[i, dsl] * (1.0 - ms)
                                   - sent_s[dsl] * ms)
                    wcb[i, bsl] = (bxb[i, dsl] * (1.0 - me)
                                   + sent_e[dsl] * me - bsb[i, dsl])
                    return carry3

                lax.fori_loop(0, HALF // 16, dim_body, 0)

            def fast_path():
                for j in range(HALF // 16):
                    dsl = pl.ds(j * 16, 16)
                    bsl = pl.ds(HALF + j * 16, 16)
                    wcb[i, dsl] = feb[i, dsl] - fsb[i, dsl]
                    wcb[i, bsl] = bxb[i, dsl] - bsb[i, dsl]

            lax.cond(has_sent, slow_path, fast_path)
            return carry2

        lax.fori_loop(0, C, span_body, 0)

    fire_gathers(0, 0)

    def pair_body(g, carry):
        for par in (0, 1):
            c = g * 2 + par

            @pl.when(c + 1 < NCHUNK)
            def _():
                fire_gathers(c + 1, 1 - par)

            wait_gathers(c, par)

            @pl.when(c >= 2)
            def _():
                src, dst, sem = out_slab(c - 2, par)
                pltpu.make_async_copy(src, dst, sem).wait()

            compute_chunk(c, par)
            src, dst, sem = out_slab(c, par)
            pltpu.async_copy(src, dst, sem)
        return carry

    lax.fori_loop(0, NCHUNK // 2, pair_body, 0)
    for c, par in ((NCHUNK - 2, 0), (NCHUNK - 1, 1)):
        src, dst, sem = out_slab(c, par)
        pltpu.make_async_copy(src, dst, sem).wait()

    for h in range(2):
        hbase = pl.multiple_of(r0 + h * 128, 128)
        cp = pltpu.async_copy(wemb_h.at[wb_idx.at[h]], wemb_buf, sem_w)
        cp.wait()

        def wrow(r, carry):
            src = wemb_buf.at[r, pl.ds(0, WIDTH_DIM)]
            dst = out.at[hbase + r, pl.ds(2 * HALF, WIDTH_DIM)]
            pltpu.async_copy(src, dst, sem_w)
            return carry

        lax.fori_loop(0, 128, wrow, 0)

        def wrow_wait(r, carry):
            src = wemb_buf.at[r, pl.ds(0, WIDTH_DIM)]
            dst = out.at[hbase + r, pl.ds(2 * HALF, WIDTH_DIM)]
            pltpu.make_async_copy(src, dst, sem_w).wait()
            return carry

        lax.fori_loop(0, 128, wrow_wait, 0)


_sc_call = functools.partial(
    pl.kernel,
    mesh=plsc.VectorSubcoreMesh(core_axis_name="c", subcore_axis_name="s"),
    out_type=jax.ShapeDtypeStruct((B * NUM_SPANS, OUT_D), jnp.float32),
    scratch_types=[
        pltpu.VMEM((C, HALF), jnp.float32),
        pltpu.VMEM((C, HALF), jnp.float32),
        pltpu.VMEM((C, HALF), jnp.float32),
        pltpu.VMEM((C, HALF), jnp.float32),
        pltpu.VMEM((C, HALF), jnp.float32),
        pltpu.VMEM((C, HALF), jnp.float32),
        pltpu.VMEM((C, HALF), jnp.float32),
        pltpu.VMEM((C, HALF), jnp.float32),
        pltpu.VMEM((C, 2 * HALF), jnp.float32),
        pltpu.VMEM((C, 2 * HALF), jnp.float32),
        pltpu.VMEM((128, 128), jnp.float32),
        pltpu.VMEM((HALF,), jnp.float32),
        pltpu.VMEM((HALF,), jnp.float32),
        pltpu.VMEM((P,), jnp.int32),
        pltpu.VMEM((P,), jnp.int32),
        pltpu.VMEM((P,), jnp.int32),
        pltpu.VMEM((P,), jnp.int32),
        pltpu.VMEM((P,), jnp.int32),
        pltpu.VMEM((P,), jnp.int32),
        pltpu.VMEM((P + 16,), jnp.float32),
        pltpu.VMEM((P + 16,), jnp.float32),
        pltpu.VMEM((2, 128), jnp.int32),
        pltpu.SemaphoreType.DMA,
        pltpu.SemaphoreType.DMA,
        pltpu.SemaphoreType.DMA,
        pltpu.SemaphoreType.DMA,
        pltpu.SemaphoreType.DMA,
    ],
)(_sc_body)


def kernel(sequence_tensor, span_indices, start_sentinel, end_sentinel,
           width_embedding):
    seq2 = sequence_tensor.reshape(B * S, D)
    starts = span_indices[..., 0].reshape(-1).astype(jnp.int32)
    ends = span_indices[..., 1].reshape(-1).astype(jnp.int32)
    wemb_p = jnp.zeros((width_embedding.shape[0], 128),
                       width_embedding.dtype).at[:, :WIDTH_DIM].set(width_embedding)
    out = _sc_call(seq2, starts, ends,
                   start_sentinel.reshape(HALF),
                   end_sentinel.reshape(HALF),
                   wemb_p)
    return out.reshape(B, NUM_SPANS, OUT_D)
